# baseline (speedup 1.0000x reference)
import functools

import jax
import jax.numpy as jnp
from jax import lax
from jax.experimental import pallas as pl
from jax.experimental.pallas import tpu as pltpu
from jax.experimental.pallas import tpu_sc as plsc

D = 64
B = 4096
L = 50
NC = 2
NS = 16
NW = NC * NS
BPW = B // NW
NCH = BPW * L // 128
RPI = 8


def _body(ids_hbm, table_hbm, pat_hbm, out_hbm,
          idx_v, trx_v, buf0, buf1, obuf, acc_sh, sem0, sem1):
    cid = lax.axis_index("c")
    sid = lax.axis_index("s")
    wid = sid * NC + cid

    pltpu.sync_copy(ids_hbm.at[pl.ds(wid * NCH, NCH)], idx_v)
    pltpu.sync_copy(pat_hbm, trx_v)

    base = sid * BPW

    def off_loop(j, carry):
        for c in range(128 // 16):
            sl = pl.ds(c * 16, 16)
            trx_v[j, sl] = trx_v[j, sl] + base
        return carry

    lax.fori_loop(0, NCH, off_loop, None)

    def zero_loop(i, carry):
        r0 = i * RPI
        z = jnp.zeros((16,), jnp.float32)
        for dr in range(RPI):
            for j in range(D // 16):
                buf0[r0 + dr, pl.ds(j * 16, 16)] = z
        return carry

    lax.fori_loop(0, BPW // RPI, zero_loop, None)
    pltpu.sync_copy(buf0, acc_sh.at[pl.ds(base, BPW)])

    cp0 = pltpu.async_copy(table_hbm.at[idx_v.at[0]], buf0, sem0)
    cp1 = pltpu.async_copy(table_hbm.at[idx_v.at[1]], buf1, sem1)

    bufs = (buf0, buf1)
    sems = (sem0, sem1)
    copies = [cp0, cp1]
    for j in range(NCH):
        b = bufs[j % 2]
        copies[j % 2].wait()
        pltpu.sync_copy(b, acc_sh.at[trx_v.at[j]], add=True)
        if j + 2 < NCH:
            copies[j % 2] = pltpu.async_copy(
                table_hbm.at[idx_v.at[j + 2]], bufs[j % 2], sems[j % 2])

    pltpu.sync_copy(acc_sh.at[pl.ds(base, BPW)], buf0)
    inv = jnp.float32(1.0 / L)

    def scale_loop(i, carry):
        r0 = i * RPI
        z = jnp.zeros((16,), jnp.float32)
        for dr in range(RPI):
            for j in range(D // 16):
                sl = pl.ds(j * 16, 16)
                obuf[r0 + dr, sl] = buf0[r0 + dr, sl] * inv
                obuf[r0 + dr, pl.ds(D + j * 16, 16)] = z
        return carry

    lax.fori_loop(0, BPW // RPI, scale_loop, None)

    pltpu.sync_copy(obuf, out_hbm.at[pl.ds(wid * BPW, BPW), :])


@jax.jit
def kernel(input_ids, table):
    ids = input_ids.astype(jnp.int32).reshape(B * L // 128, 128)
    pat = (jnp.arange(BPW * L, dtype=jnp.int32) // L).reshape(NCH, 128)
    mesh = plsc.VectorSubcoreMesh(core_axis_name="c", subcore_axis_name="s")
    k = functools.partial(
        pl.kernel,
        mesh=mesh,
        out_type=jax.ShapeDtypeStruct((B, 128), jnp.float32),
        scratch_types=[
            pltpu.VMEM((NCH, 128), jnp.int32),
            pltpu.VMEM((NCH, 128), jnp.int32),
            pltpu.VMEM((BPW, D), jnp.float32),
            pltpu.VMEM((BPW, D), jnp.float32),
            pltpu.VMEM((BPW, 128), jnp.float32),
            pltpu.VMEM_SHARED((NS * BPW, D), jnp.float32),
            pltpu.SemaphoreType.DMA,
            pltpu.SemaphoreType.DMA,
        ],
        compiler_params=pltpu.CompilerParams(use_tc_tiling_on_sc=False),
    )(_body)
    return k(ids, table, pat)[:, :D]


# speedup vs baseline: 1.0282x; 1.0282x over previous
import functools

import jax
import jax.numpy as jnp
from jax import lax
from jax.experimental import pallas as pl
from jax.experimental.pallas import tpu as pltpu
from jax.experimental.pallas import tpu_sc as plsc

D = 64
B = 4096
L = 50
NC = 2
NS = 16
NW = NC * NS
BPW = B // NW
NCH = BPW * L // 128
RPI = 8


NBUF = 4


def _body(ids_hbm, table_hbm, pat_hbm, out_hbm,
          idx_v, trx_v, buf0, buf1, buf2, buf3, obuf, acc_sh,
          g0, g1, g2, g3, s0, s1, s2, s3):
    cid = lax.axis_index("c")
    sid = lax.axis_index("s")
    wid = sid * NC + cid

    pltpu.sync_copy(ids_hbm.at[pl.ds(wid * NCH, NCH)], idx_v)
    pltpu.sync_copy(pat_hbm, trx_v)

    base = sid * BPW

    def off_loop(j, carry):
        for c in range(128 // 16):
            sl = pl.ds(c * 16, 16)
            trx_v[j, sl] = trx_v[j, sl] + base
        return carry

    lax.fori_loop(0, NCH, off_loop, None)

    def zero_loop(i, carry):
        r0 = i * RPI
        z = jnp.zeros((16,), jnp.float32)
        for dr in range(RPI):
            for j in range(D // 16):
                buf0[r0 + dr, pl.ds(j * 16, 16)] = z
        return carry

    lax.fori_loop(0, BPW // RPI, zero_loop, None)
    pltpu.sync_copy(buf0, acc_sh.at[pl.ds(base, BPW)])

    bufs = (buf0, buf1, buf2, buf3)
    gsems = (g0, g1, g2, g3)
    ssems = (s0, s1, s2, s3)

    copies = [
        pltpu.async_copy(table_hbm.at[idx_v.at[j]], bufs[j], gsems[j])
        for j in range(NBUF)
    ]
    scats = [None] * NBUF
    for j in range(NCH):
        b = j % NBUF
        copies[b].wait()
        if j >= 1 and j - 1 + NBUF < NCH:
            pb = (j - 1) % NBUF
            scats[pb].wait()
            scats[pb] = None
            copies[pb] = pltpu.async_copy(
                table_hbm.at[idx_v.at[j - 1 + NBUF]], bufs[pb], gsems[pb])
        scats[b] = pltpu.async_copy(
            bufs[b], acc_sh.at[trx_v.at[j]], ssems[b], add=True)
    for b in range(NBUF):
        if scats[b] is not None:
            scats[b].wait()

    pltpu.sync_copy(acc_sh.at[pl.ds(base, BPW)], buf0)
    inv = jnp.float32(1.0 / L)

    def scale_loop(i, carry):
        r0 = i * RPI
        z = jnp.zeros((16,), jnp.float32)
        for dr in range(RPI):
            for j in range(D // 16):
                sl = pl.ds(j * 16, 16)
                obuf[r0 + dr, sl] = buf0[r0 + dr, sl] * inv
                obuf[r0 + dr, pl.ds(D + j * 16, 16)] = z
        return carry

    lax.fori_loop(0, BPW // RPI, scale_loop, None)

    pltpu.sync_copy(obuf, out_hbm.at[pl.ds(wid * BPW, BPW), :])


@jax.jit
def kernel(input_ids, table):
    ids = input_ids.astype(jnp.int32).reshape(B * L // 128, 128)
    pat = (jnp.arange(BPW * L, dtype=jnp.int32) // L).reshape(NCH, 128)
    mesh = plsc.VectorSubcoreMesh(core_axis_name="c", subcore_axis_name="s")
    k = functools.partial(
        pl.kernel,
        mesh=mesh,
        out_type=jax.ShapeDtypeStruct((B, 128), jnp.float32),
        scratch_types=[
            pltpu.VMEM((NCH, 128), jnp.int32),
            pltpu.VMEM((NCH, 128), jnp.int32),
            pltpu.VMEM((BPW, D), jnp.float32),
            pltpu.VMEM((BPW, D), jnp.float32),
            pltpu.VMEM((BPW, D), jnp.float32),
            pltpu.VMEM((BPW, D), jnp.float32),
            pltpu.VMEM((BPW, 128), jnp.float32),
            pltpu.VMEM_SHARED((NS * BPW, D), jnp.float32),
        ] + [pltpu.SemaphoreType.DMA] * (2 * NBUF),
        compiler_params=pltpu.CompilerParams(use_tc_tiling_on_sc=False),
    )(_body)
    return k(ids, table, pat)[:, :D]


# speedup vs baseline: 1.0436x; 1.0150x over previous
import functools

import jax
import jax.numpy as jnp
from jax import lax
from jax.experimental import pallas as pl
from jax.experimental.pallas import tpu as pltpu
from jax.experimental.pallas import tpu_sc as plsc

D = 64
B = 4096
L = 50
NC = 2
NS = 16
NW = NC * NS
BPW = B // NW
NCH = BPW * L // 128
RPI = 8
NBUF = 4


def _body(ids_hbm, table_hbm, pat_hbm, out_hbm,
          idx_v, trx_v, buf0, buf1, buf2, buf3, obuf, zbuf, acc_sh,
          g0, g1, g2, g3, s0, s1, s2, s3):
    cid = lax.axis_index("c")
    sid = lax.axis_index("s")
    wid = sid * NC + cid

    pltpu.sync_copy(ids_hbm.at[pl.ds(wid * NCH, NCH)], idx_v)

    bufs = (buf0, buf1, buf2, buf3)
    gsems = (g0, g1, g2, g3)
    ssems = (s0, s1, s2, s3)
    copies = [
        pltpu.async_copy(table_hbm.at[idx_v.at[j]], bufs[j], gsems[j])
        for j in range(NBUF)
    ]

    pltpu.sync_copy(pat_hbm, trx_v)
    base = sid * BPW

    def off_loop(j, carry):
        for c in range(128 // 16):
            sl = pl.ds(c * 16, 16)
            trx_v[j, sl] = trx_v[j, sl] + base
        return carry

    lax.fori_loop(0, NCH, off_loop, None)

    def zero_loop(i, carry):
        r0 = i * RPI
        z = jnp.zeros((16,), jnp.float32)
        for dr in range(RPI):
            for j in range(D // 16):
                zbuf[r0 + dr, pl.ds(j * 16, 16)] = z
        return carry

    lax.fori_loop(0, BPW // RPI, zero_loop, None)
    pltpu.sync_copy(zbuf, acc_sh.at[pl.ds(base, BPW)])

    scats = [None] * NBUF
    for j in range(NCH):
        b = j % NBUF
        copies[b].wait()
        if j >= 1 and j - 1 + NBUF < NCH:
            pb = (j - 1) % NBUF
            scats[pb].wait()
            scats[pb] = None
            copies[pb] = pltpu.async_copy(
                table_hbm.at[idx_v.at[j - 1 + NBUF]], bufs[pb], gsems[pb])
        scats[b] = pltpu.async_copy(
            bufs[b], acc_sh.at[trx_v.at[j]], ssems[b], add=True)
    for b in range(NBUF):
        if scats[b] is not None:
            scats[b].wait()

    pltpu.sync_copy(acc_sh.at[pl.ds(base, BPW)], zbuf)
    inv = jnp.float32(1.0 / L)

    def scale_loop(i, carry):
        r0 = i * RPI
        z = jnp.zeros((16,), jnp.float32)
        for dr in range(RPI):
            for j in range(D // 16):
                sl = pl.ds(j * 16, 16)
                obuf[r0 + dr, sl] = zbuf[r0 + dr, sl] * inv
                obuf[r0 + dr, pl.ds(D + j * 16, 16)] = z
        return carry

    lax.fori_loop(0, BPW // RPI, scale_loop, None)

    pltpu.sync_copy(obuf, out_hbm.at[pl.ds(wid * BPW, BPW), :])


@jax.jit
def kernel(input_ids, table):
    ids = input_ids.astype(jnp.int32).reshape(B * L // 128, 128)
    pat = (jnp.arange(BPW * L, dtype=jnp.int32) // L).reshape(NCH, 128)
    mesh = plsc.VectorSubcoreMesh(core_axis_name="c", subcore_axis_name="s")
    k = functools.partial(
        pl.kernel,
        mesh=mesh,
        out_type=jax.ShapeDtypeStruct((B, 128), jnp.float32),
        scratch_types=[
            pltpu.VMEM((NCH, 128), jnp.int32),
            pltpu.VMEM((NCH, 128), jnp.int32),
            pltpu.VMEM((BPW, D), jnp.float32),
            pltpu.VMEM((BPW, D), jnp.float32),
            pltpu.VMEM((BPW, D), jnp.float32),
            pltpu.VMEM((BPW, D), jnp.float32),
            pltpu.VMEM((BPW, 128), jnp.float32),
            pltpu.VMEM((BPW, D), jnp.float32),
            pltpu.VMEM_SHARED((NS * BPW, D), jnp.float32),
        ] + [pltpu.SemaphoreType.DMA] * (2 * NBUF),
        compiler_params=pltpu.CompilerParams(use_tc_tiling_on_sc=False),
    )(_body)
    return k(ids, table, pat)[:, :D]
